# initial kernel scaffold (unmeasured)
import jax
import jax.numpy as jnp
from jax import lax
from jax.experimental import pallas as pl
from jax.experimental.pallas import tpu as pltpu


def kernel(
    x,
):
    def body(*refs):
        pass

    out_shape = jax.ShapeDtypeStruct(..., jnp.float32)
    return pl.pallas_call(body, out_shape=out_shape)(...)


# baseline (device time: 152114 ns/iter reference)
import jax
import jax.numpy as jnp
from jax import lax
from jax.experimental import pallas as pl
from jax.experimental.pallas import tpu as pltpu

M = 4096
N = 2048
HALF = N // 2
BLK = M // 8
MH = M // 2


def kernel(x):
    def body(x_ref, out_ref, send_sems, recv_sems):
        p = lax.axis_index("x")
        q = lax.axis_index("y")
        r = lax.axis_index("z")

        barrier = pltpu.get_barrier_semaphore()
        pl.semaphore_signal(barrier, inc=1, device_id=(1 - p, q, r),
                            device_id_type=pl.DeviceIdType.MESH)
        pl.semaphore_signal(barrier, inc=1, device_id=(p, 1 - q, r),
                            device_id_type=pl.DeviceIdType.MESH)
        pl.semaphore_signal(barrier, inc=1, device_id=(p, q, 1 - r),
                            device_id_type=pl.DeviceIdType.MESH)
        pl.semaphore_wait(barrier, 3)

        a_me = 2 * r + q
        b_me = 2 * q + r

        my_cols = pl.ds(p * HALF, HALF)
        nbr_cols = pl.ds((1 - p) * HALF, HALF)

        rows_a = pl.ds(a_me * BLK, BLK)
        rows_b = pl.ds(MH + b_me * BLK, BLK)

        p1a = pltpu.make_async_remote_copy(
            src_ref=x_ref.at[0, rows_a, nbr_cols],
            dst_ref=out_ref.at[rows_a, :],
            send_sem=send_sems.at[0], recv_sem=recv_sems.at[0],
            device_id=(1 - p, q, r), device_id_type=pl.DeviceIdType.MESH)
        p1b = pltpu.make_async_remote_copy(
            src_ref=x_ref.at[0, rows_b, nbr_cols],
            dst_ref=out_ref.at[rows_b, :],
            send_sem=send_sems.at[1], recv_sem=recv_sems.at[1],
            device_id=(1 - p, q, r), device_id_type=pl.DeviceIdType.MESH)
        p1a.start()
        p1b.start()
        p1a.wait()
        p1b.wait()

        p2y = pltpu.make_async_remote_copy(
            src_ref=out_ref.at[rows_a, :],
            dst_ref=out_ref.at[rows_a, :],
            send_sem=send_sems.at[2], recv_sem=recv_sems.at[2],
            device_id=(p, 1 - q, r), device_id_type=pl.DeviceIdType.MESH)
        p2z = pltpu.make_async_remote_copy(
            src_ref=out_ref.at[rows_b, :],
            dst_ref=out_ref.at[rows_b, :],
            send_sem=send_sems.at[3], recv_sem=recv_sems.at[3],
            device_id=(p, q, 1 - r), device_id_type=pl.DeviceIdType.MESH)
        p2y.start()
        p2z.start()
        p2y.wait()
        p2z.wait()

        rows_a2 = pl.ds(2 * r * BLK, 2 * BLK)
        rows_b2 = pl.ds(MH + 2 * q * BLK, 2 * BLK)
        p3z = pltpu.make_async_remote_copy(
            src_ref=out_ref.at[rows_a2, :],
            dst_ref=out_ref.at[rows_a2, :],
            send_sem=send_sems.at[4], recv_sem=recv_sems.at[4],
            device_id=(p, q, 1 - r), device_id_type=pl.DeviceIdType.MESH)
        p3y = pltpu.make_async_remote_copy(
            src_ref=out_ref.at[rows_b2, :],
            dst_ref=out_ref.at[rows_b2, :],
            send_sem=send_sems.at[5], recv_sem=recv_sems.at[5],
            device_id=(p, 1 - q, r), device_id_type=pl.DeviceIdType.MESH)
        p3z.start()
        p3y.start()
        p3z.wait()
        p3y.wait()

        out_ref[:, :] = out_ref[:, :] + x_ref[0, :, my_cols]

    return pl.pallas_call(
        body,
        out_shape=jax.ShapeDtypeStruct((M, HALF), jnp.float32),
        in_specs=[pl.BlockSpec(memory_space=pltpu.VMEM)],
        out_specs=pl.BlockSpec(memory_space=pltpu.VMEM),
        scratch_shapes=[
            pltpu.SemaphoreType.DMA((6,)),
            pltpu.SemaphoreType.DMA((6,)),
        ],
        compiler_params=pltpu.CompilerParams(
            collective_id=0,
            vmem_limit_bytes=60 * 1024 * 1024,
        ),
    )(x)


# device time: 133184 ns/iter; 1.1421x vs baseline; 1.1421x over previous
import jax
import jax.numpy as jnp
from jax import lax
from jax.experimental import pallas as pl
from jax.experimental.pallas import tpu as pltpu

M = 4096
N = 2048
HALF = N // 2
BLK = M // 8
MH = M // 2
CH = 4
CW = HALF // CH


def kernel(x):
    def body(x_ref, out_ref, p3buf, s1, r1, s2, r2, s3, r3):
        p = lax.axis_index("x")
        q = lax.axis_index("y")
        r = lax.axis_index("z")

        barrier = pltpu.get_barrier_semaphore()
        pl.semaphore_signal(barrier, inc=1, device_id=(1 - p, q, r),
                            device_id_type=pl.DeviceIdType.MESH)
        pl.semaphore_signal(barrier, inc=1, device_id=(p, 1 - q, r),
                            device_id_type=pl.DeviceIdType.MESH)
        pl.semaphore_signal(barrier, inc=1, device_id=(p, q, 1 - r),
                            device_id_type=pl.DeviceIdType.MESH)
        pl.semaphore_wait(barrier, 3)

        a_me = 2 * r + q
        b_me = 2 * q + r

        rows_a = pl.ds(a_me * BLK, BLK)
        rows_b = pl.ds(MH + b_me * BLK, BLK)
        rows_a2 = pl.ds(2 * r * BLK, 2 * BLK)
        rows_b2 = pl.ds(MH + 2 * q * BLK, 2 * BLK)
        rows_a3 = pl.ds(2 * (1 - r) * BLK, 2 * BLK)
        rows_b3 = pl.ds(MH + 2 * (1 - q) * BLK, 2 * BLK)

        def oc(c):
            return pl.ds(c * CW, CW)

        def mc(c):
            return pl.ds(p * HALF + c * CW, CW)

        def nc(c):
            return pl.ds((1 - p) * HALF + c * CW, CW)

        p1a = []
        p1b = []
        for c in range(CH):
            da = pltpu.make_async_remote_copy(
                src_ref=x_ref.at[0, rows_a, nc(c)],
                dst_ref=out_ref.at[rows_a, oc(c)],
                send_sem=s1.at[0, c], recv_sem=r1.at[0, c],
                device_id=(1 - p, q, r), device_id_type=pl.DeviceIdType.MESH)
            db = pltpu.make_async_remote_copy(
                src_ref=x_ref.at[0, rows_b, nc(c)],
                dst_ref=out_ref.at[rows_b, oc(c)],
                send_sem=s1.at[1, c], recv_sem=r1.at[1, c],
                device_id=(1 - p, q, r), device_id_type=pl.DeviceIdType.MESH)
            da.start()
            db.start()
            p1a.append(da)
            p1b.append(db)

        out_ref[rows_a3, :] = x_ref[0, rows_a3, pl.ds(p * HALF, HALF)]
        out_ref[rows_b3, :] = x_ref[0, rows_b3, pl.ds(p * HALF, HALF)]

        p2y = []
        p2z = []
        for c in range(CH):
            p1a[c].wait()
            p1b[c].wait()
            dy = pltpu.make_async_remote_copy(
                src_ref=out_ref.at[rows_a, oc(c)],
                dst_ref=out_ref.at[rows_a, oc(c)],
                send_sem=s2.at[0, c], recv_sem=r2.at[0, c],
                device_id=(p, 1 - q, r), device_id_type=pl.DeviceIdType.MESH)
            dz = pltpu.make_async_remote_copy(
                src_ref=out_ref.at[rows_b, oc(c)],
                dst_ref=out_ref.at[rows_b, oc(c)],
                send_sem=s2.at[1, c], recv_sem=r2.at[1, c],
                device_id=(p, q, 1 - r), device_id_type=pl.DeviceIdType.MESH)
            dy.start()
            dz.start()
            p2y.append(dy)
            p2z.append(dz)

        p3z = []
        p3y = []
        for c in range(CH):
            p2y[c].wait()
            p2z[c].wait()
            dz = pltpu.make_async_remote_copy(
                src_ref=out_ref.at[rows_a2, oc(c)],
                dst_ref=p3buf.at[pl.ds(0, 2 * BLK), oc(c)],
                send_sem=s3.at[0, c], recv_sem=r3.at[0, c],
                device_id=(p, q, 1 - r), device_id_type=pl.DeviceIdType.MESH)
            dy = pltpu.make_async_remote_copy(
                src_ref=out_ref.at[rows_b2, oc(c)],
                dst_ref=p3buf.at[pl.ds(2 * BLK, 2 * BLK), oc(c)],
                send_sem=s3.at[1, c], recv_sem=r3.at[1, c],
                device_id=(p, 1 - q, r), device_id_type=pl.DeviceIdType.MESH)
            dz.start()
            dy.start()
            p3z.append(dz)
            p3y.append(dy)

        for c in range(CH):
            p3z[c].wait_send()
            p3y[c].wait_send()
            out_ref[rows_a2, oc(c)] = (
                out_ref[rows_a2, oc(c)] + x_ref[0, rows_a2, mc(c)])
            out_ref[rows_b2, oc(c)] = (
                out_ref[rows_b2, oc(c)] + x_ref[0, rows_b2, mc(c)])

        for c in range(CH):
            p3z[c].wait_recv()
            p3y[c].wait_recv()
            out_ref[rows_a3, oc(c)] = (
                out_ref[rows_a3, oc(c)] + p3buf[pl.ds(0, 2 * BLK), oc(c)])
            out_ref[rows_b3, oc(c)] = (
                out_ref[rows_b3, oc(c)] + p3buf[pl.ds(2 * BLK, 2 * BLK), oc(c)])

    return pl.pallas_call(
        body,
        out_shape=jax.ShapeDtypeStruct((M, HALF), jnp.float32),
        in_specs=[pl.BlockSpec(memory_space=pltpu.VMEM)],
        out_specs=pl.BlockSpec(memory_space=pltpu.VMEM),
        scratch_shapes=[
            pltpu.VMEM((4 * BLK, HALF), jnp.float32),
            pltpu.SemaphoreType.DMA((2, CH)),
            pltpu.SemaphoreType.DMA((2, CH)),
            pltpu.SemaphoreType.DMA((2, CH)),
            pltpu.SemaphoreType.DMA((2, CH)),
            pltpu.SemaphoreType.DMA((2, CH)),
            pltpu.SemaphoreType.DMA((2, CH)),
        ],
        compiler_params=pltpu.CompilerParams(
            collective_id=0,
            vmem_limit_bytes=60 * 1024 * 1024,
        ),
    )(x)


# device time: 105663 ns/iter; 1.4396x vs baseline; 1.2605x over previous
import jax
import jax.numpy as jnp
from jax import lax
from jax.experimental import pallas as pl
from jax.experimental.pallas import tpu as pltpu

M = 4096
N = 2048
HALF = N // 2
CH = 8
CW = HALF // CH

EQ = 656
EH = EQ // 2
DSTART = 4 * EQ
DP = 736
DH = DP // 2


def kernel(x):
    def body(x_ref, out_ref, ssem, rsem):
        p = lax.axis_index("x")
        q = lax.axis_index("y")
        r = lax.axis_index("z")

        barrier = pltpu.get_barrier_semaphore()
        pl.semaphore_signal(barrier, inc=1, device_id=(1 - p, q, r),
                            device_id_type=pl.DeviceIdType.MESH)
        pl.semaphore_signal(barrier, inc=1, device_id=(p, 1 - q, r),
                            device_id_type=pl.DeviceIdType.MESH)
        pl.semaphore_signal(barrier, inc=1, device_id=(p, q, 1 - r),
                            device_id_type=pl.DeviceIdType.MESH)
        pl.semaphore_wait(barrier, 3)

        k_me = 2 * q + r
        k_y = 2 * (1 - q) + r
        k_z = 2 * q + (1 - r)
        pair = (q + r) % 2
        d_me = DSTART + pair * DP

        e_rows = pl.ds(k_me * EQ, EQ)
        d_rows = pl.ds(d_me, DP)
        d1_rows = pl.ds(d_me, DH)
        d2_rows = pl.ds(d_me + DH, DH)
        fz_rows = pl.ds(k_y * EQ, EH)
        fy_rows = pl.ds(k_z * EQ + EH, EH)

        def oc(c):
            return pl.ds(c * CW, CW)

        def mc(c):
            return pl.ds(p * HALF + c * CW, CW)

        def nc(c):
            return pl.ds((1 - p) * HALF + c * CW, CW)

        X_NBR = (1 - p, q, r)
        Y_NBR = (p, 1 - q, r)
        Z_NBR = (p, q, 1 - r)

        def rdma(slot, c, src, dst, dev):
            return pltpu.make_async_remote_copy(
                src_ref=src, dst_ref=dst,
                send_sem=ssem.at[slot, c], recv_sem=rsem.at[slot, c],
                device_id=dev, device_id_type=pl.DeviceIdType.MESH)

        e1_rows = pl.ds(k_me * EQ, EH)
        e2_rows = pl.ds(k_me * EQ + EH, EH)

        xe1 = []
        xe2 = []
        xd = []
        for c in range(CH):
            d1 = rdma(0, c, x_ref.at[0, e1_rows, nc(c)],
                      out_ref.at[e1_rows, oc(c)], X_NBR)
            d2 = rdma(1, c, x_ref.at[0, e2_rows, nc(c)],
                      out_ref.at[e2_rows, oc(c)], X_NBR)
            db = rdma(2, c, x_ref.at[0, d_rows, nc(c)],
                      out_ref.at[d_rows, oc(c)], X_NBR)
            d1.start()
            d2.start()
            db.start()
            xe1.append(d1)
            xe2.append(d2)
            xd.append(db)

        ye1 = [None] * CH
        ye2 = [None] * CH
        ze1 = [None] * CH
        ze2 = [None] * CH
        yd = [None] * CH
        zd = [None] * CH
        yf = [None] * CH
        zf = [None] * CH

        def fire_forwards(c):
            ye1[c].wait()
            zf[c] = rdma(9, c, out_ref.at[fz_rows, oc(c)],
                         out_ref.at[fz_rows, oc(c)], Z_NBR)
            zf[c].start()
            ze2[c].wait()
            yf[c] = rdma(8, c, out_ref.at[fy_rows, oc(c)],
                         out_ref.at[fy_rows, oc(c)], Y_NBR)
            yf[c].start()

        for c in range(CH):
            xe1[c].wait()
            ye1[c] = rdma(3, c, out_ref.at[e1_rows, oc(c)],
                          out_ref.at[e1_rows, oc(c)], Y_NBR)
            ze1[c] = rdma(4, c, out_ref.at[e1_rows, oc(c)],
                          out_ref.at[e1_rows, oc(c)], Z_NBR)
            ye1[c].start()
            ze1[c].start()
            xe2[c].wait()
            ye2[c] = rdma(5, c, out_ref.at[e2_rows, oc(c)],
                          out_ref.at[e2_rows, oc(c)], Y_NBR)
            ze2[c] = rdma(6, c, out_ref.at[e2_rows, oc(c)],
                          out_ref.at[e2_rows, oc(c)], Z_NBR)
            ye2[c].start()
            ze2[c].start()
            xd[c].wait()
            yd[c] = rdma(7, c, out_ref.at[d1_rows, oc(c)],
                         out_ref.at[d1_rows, oc(c)], Y_NBR)
            zd[c] = rdma(10, c, out_ref.at[d2_rows, oc(c)],
                         out_ref.at[d2_rows, oc(c)], Z_NBR)
            yd[c].start()
            zd[c].start()
            if c >= 1:
                fire_forwards(c - 1)
        fire_forwards(CH - 1)

        for c in range(CH):
            ye2[c].wait()
            ze1[c].wait()
            yd[c].wait()
            zd[c].wait()
            yf[c].wait()
            zf[c].wait()
            out_ref[:, oc(c)] = out_ref[:, oc(c)] + x_ref[0, :, mc(c)]

    return pl.pallas_call(
        body,
        out_shape=jax.ShapeDtypeStruct((M, HALF), jnp.float32),
        in_specs=[pl.BlockSpec(memory_space=pltpu.VMEM)],
        out_specs=pl.BlockSpec(memory_space=pltpu.VMEM),
        scratch_shapes=[
            pltpu.SemaphoreType.DMA((11, CH)),
            pltpu.SemaphoreType.DMA((11, CH)),
        ],
        compiler_params=pltpu.CompilerParams(
            collective_id=0,
            vmem_limit_bytes=60 * 1024 * 1024,
        ),
    )(x)
